# baseline (speedup 1.0000x reference)
import functools
import math

import jax
import jax.numpy as jnp
from jax import lax
from jax.experimental import pallas as pl
from jax.experimental.pallas import tpu as pltpu
from jax.experimental.pallas import tpu_sc as plsc

N_NODES = 50000
NP = 50176
RB = 512
GRID = NP // RB
NG = 8
TDIM = 128
EPS = 1e-6


def _silu(x):
    return x * (1.0 / (1.0 + jnp.exp(-x)))


def _mm_t(x, w, hi=False):
    return lax.dot_general(x, w, (((1,), (1,)), ((), ())),
                           precision=lax.Precision.HIGHEST if hi else None,
                           preferred_element_type=jnp.float32)


def _onehot(b, rows):
    return (b == lax.broadcasted_iota(jnp.int32, (rows, NG), 1)).astype(jnp.float32)


def _pre_body(t_ref, w1_ref, w2_ref, *rest):
    n_layers = (len(rest) - 1) // 3
    tws = rest[:n_layers]
    tbs = rest[n_layers:2 * n_layers]
    cnt_in = rest[2 * n_layers]
    outs = rest[2 * n_layers + 1:]
    half = TDIM // 2
    k = lax.broadcasted_iota(jnp.int32, (half, 1), 0).astype(jnp.float32)
    emb = jnp.exp(k * (-math.log(10000.0) / (half - 1)))
    e = _mm_t(t_ref[...], emb, hi=True)
    e2 = jnp.concatenate([jnp.sin(e), jnp.cos(e)], axis=-1)
    tv = _mm_t(_silu(_mm_t(e2, w1_ref[...], hi=True)), w2_ref[...], hi=True)
    st = _silu(tv)
    for i in range(n_layers):
        outs[i][...] = _mm_t(st, tws[i][...], hi=True) + tbs[i][...]
    del cnt_in


def _k_pre(t2, w1, w2, tws, tbs):
    nl = len(tws)
    spec_full = lambda a: pl.BlockSpec(a.shape, lambda: tuple(0 for _ in a.shape))
    in_arrays = [t2, w1, w2] + list(tws) + list(tbs) + [jnp.zeros((1, 1), jnp.float32)]
    out_shapes = tuple(jax.ShapeDtypeStruct((NG, tw.shape[0]), jnp.float32) for tw in tws)
    return pl.pallas_call(
        _pre_body,
        out_shape=out_shapes,
        in_specs=[spec_full(a) for a in in_arrays],
        out_specs=tuple(pl.BlockSpec(s.shape, lambda: (0, 0)) for s in out_shapes),
    )(*in_arrays)


def _in_body(xin_ref, batch_ref, inw_ref, y2_ref, sums_ref, cnt_ref):
    x = _mm_t(xin_ref[...], inw_ref[...], hi=True)
    c2 = x.shape[1] // 2
    y2_ref[0] = x[:, :c2]
    y2_ref[1] = x[:, c2:]
    g = _onehot(batch_ref[...], x.shape[0])

    @pl.when(pl.program_id(0) == 0)
    def _():
        sums_ref[...] = jnp.zeros_like(sums_ref)
        cnt_ref[...] = jnp.zeros_like(cnt_ref)

    sums_ref[...] += lax.dot_general(g, x * x, (((0,), (0,)), ((), ())),
                                     precision=lax.Precision.HIGHEST,
                                     preferred_element_type=jnp.float32)
    cnt_ref[...] += jnp.sum(g, axis=0)[:, None]


def _k_in(xin_p, batch_p, in_w):
    ch = in_w.shape[0]
    return pl.pallas_call(
        _in_body,
        grid=(GRID,),
        out_shape=(
            jax.ShapeDtypeStruct((2, NP, ch // 2), jnp.float32),
            jax.ShapeDtypeStruct((NG, ch), jnp.float32),
            jax.ShapeDtypeStruct((NG, 1), jnp.float32),
        ),
        in_specs=[
            pl.BlockSpec((RB, 2), lambda i: (i, 0)),
            pl.BlockSpec((RB, 1), lambda i: (i, 0)),
            pl.BlockSpec(in_w.shape, lambda i: (0, 0)),
        ],
        out_specs=(
            pl.BlockSpec((2, RB, ch // 2), lambda i: (0, i, 0)),
            pl.BlockSpec((NG, ch), lambda i: (0, 0)),
            pl.BlockSpec((NG, 1), lambda i: (0, 0)),
        ),
    )(xin_p, batch_p, in_w)


def _a_body(has_mlp, x2_ref, batch_ref, inv8_ref, style_ref, field_ref,
            fwt_ref, *mlp_refs):
    x = jnp.concatenate([x2_ref[0], x2_ref[1]], axis=-1)
    ch = x.shape[1]
    g = _onehot(batch_ref[...], x.shape[0])
    xn = x * jnp.dot(g, inv8_ref[...], precision=lax.Precision.HIGHEST, preferred_element_type=jnp.float32)
    cond = jnp.dot(g, style_ref[...], precision=lax.Precision.HIGHEST, preferred_element_type=jnp.float32) \
        + _mm_t(field_ref[...], fwt_ref[...], hi=True)
    xf = _silu(xn * (1.0 + cond[:, :ch]) + cond[:, ch:])
    if has_mlp:
        m0w_ref, m0b_ref, m1w_ref, m1b_ref, y2_ref = mlp_refs
        h = _mm_t(_silu(_mm_t(xf, m0w_ref[...]) + m0b_ref[...]), m1w_ref[...]) \
            + m1b_ref[...]
        xf = xf + h
    else:
        (y2_ref,) = mlp_refs
    c2 = ch // 2
    y2_ref[0] = xf[:, :c2]
    y2_ref[1] = xf[:, c2:]


def _k_a(x2, batch_p, inv8, style, field_p, p):
    ch = x2.shape[2] * 2
    has_mlp = "m0_w" in p
    fwt = p["f_w"]
    extra = []
    extra_specs = []
    if has_mlp:
        extra = [p["m0_w"], p["m0_b"].reshape(1, -1), p["m1_w"], p["m1_b"].reshape(1, -1)]
        extra_specs = [pl.BlockSpec(a.shape, lambda i: (0, 0)) for a in extra]
    return pl.pallas_call(
        functools.partial(_a_body, has_mlp),
        grid=(GRID,),
        out_shape=jax.ShapeDtypeStruct((2, NP, ch // 2), jnp.float32),
        in_specs=[
            pl.BlockSpec((2, RB, ch // 2), lambda i: (0, i, 0)),
            pl.BlockSpec((RB, 1), lambda i: (i, 0)),
            pl.BlockSpec(inv8.shape, lambda i: (0, 0)),
            pl.BlockSpec(style.shape, lambda i: (0, 0)),
            pl.BlockSpec((RB, 1), lambda i: (i, 0)),
            pl.BlockSpec(fwt.shape, lambda i: (0, 0)),
        ] + extra_specs,
        out_specs=pl.BlockSpec((2, RB, ch // 2), lambda i: (0, i, 0)),
    )(x2, batch_p, inv8, style, field_p, fwt, *extra)


def _b_body(agg_ref, y_ref, rel_ref, root_ref, out_ref):
    agg = jnp.concatenate([agg_ref[0], agg_ref[1]], axis=-1)
    y = jnp.concatenate([y_ref[0], y_ref[1]], axis=-1)
    xm = _silu(_mm_t(agg, rel_ref[...]) + _mm_t(y, root_ref[...]))
    c2 = xm.shape[1] // 2
    out_ref[0] = xm[:, :c2]
    out_ref[1] = xm[:, c2:]


def _k_b(agg2, y2, rel, root):
    ch = y2.shape[2] * 2
    return pl.pallas_call(
        _b_body,
        grid=(GRID,),
        out_shape=jax.ShapeDtypeStruct((2, NP, ch // 2), jnp.float32),
        in_specs=[
            pl.BlockSpec((2, RB, ch // 2), lambda i: (0, i, 0)),
            pl.BlockSpec((2, RB, ch // 2), lambda i: (0, i, 0)),
            pl.BlockSpec(rel.shape, lambda i: (0, 0)),
            pl.BlockSpec(root.shape, lambda i: (0, 0)),
        ],
        out_specs=pl.BlockSpec((2, RB, ch // 2), lambda i: (0, i, 0)),
    )(agg2, y2, rel, root)


def _c_body(has_sc, has_res, agg_ref, xm_ref, xin_ref, batch_ref, rel_ref,
            root_ref, *rest):
    idx = 0
    sc_ref = None
    res_ref = None
    if has_sc:
        sc_ref = rest[idx]; idx += 1
    if has_res:
        res_ref = rest[idx]; idx += 1
    out2_ref, sums_ref, cnt_ref = rest[idx:]

    agg = jnp.concatenate([agg_ref[0], agg_ref[1]], axis=-1)
    xm = jnp.concatenate([xm_ref[0], xm_ref[1]], axis=-1)
    xin = jnp.concatenate([xin_ref[0], xin_ref[1]], axis=-1)
    out = _mm_t(agg, rel_ref[...]) + _mm_t(xm, root_ref[...])
    if has_sc:
        out = out + _mm_t(xin, sc_ref[...])
    else:
        out = out + xin
    if has_res:
        out = out + jnp.concatenate([res_ref[0], res_ref[1]], axis=-1)

    c2 = out.shape[1] // 2
    out2_ref[0] = out[:, :c2]
    out2_ref[1] = out[:, c2:]

    g = _onehot(batch_ref[...], out.shape[0])

    @pl.when(pl.program_id(0) == 0)
    def _():
        sums_ref[...] = jnp.zeros_like(sums_ref)
        cnt_ref[...] = jnp.zeros_like(cnt_ref)

    sums_ref[...] += lax.dot_general(g, out * out, (((0,), (0,)), ((), ())),
                                     precision=lax.Precision.HIGHEST,
                                     preferred_element_type=jnp.float32)
    cnt_ref[...] += jnp.sum(g, axis=0)[:, None]


def _k_c(agg2, xm2, xin2, batch_p, p, res2=None):
    rel, root = p["c1_rel"], p["c1_root"]
    out_ch = rel.shape[0]
    in_c2 = xm2.shape[2]
    has_sc = "sc_w" in p
    extra = []
    extra_specs = []
    if has_sc:
        extra.append(p["sc_w"])
        extra_specs.append(pl.BlockSpec(p["sc_w"].shape, lambda i: (0, 0)))
    if res2 is not None:
        extra.append(res2)
        extra_specs.append(pl.BlockSpec((2, RB, res2.shape[2]), lambda i: (0, i, 0)))
    return pl.pallas_call(
        functools.partial(_c_body, has_sc, res2 is not None),
        grid=(GRID,),
        out_shape=(
            jax.ShapeDtypeStruct((2, NP, out_ch // 2), jnp.float32),
            jax.ShapeDtypeStruct((NG, out_ch), jnp.float32),
            jax.ShapeDtypeStruct((NG, 1), jnp.float32),
        ),
        in_specs=[
            pl.BlockSpec((2, RB, in_c2), lambda i: (0, i, 0)),
            pl.BlockSpec((2, RB, in_c2), lambda i: (0, i, 0)),
            pl.BlockSpec((2, RB, xin2.shape[2]), lambda i: (0, i, 0)),
            pl.BlockSpec((RB, 1), lambda i: (i, 0)),
            pl.BlockSpec(rel.shape, lambda i: (0, 0)),
            pl.BlockSpec(root.shape, lambda i: (0, 0)),
        ] + extra_specs,
        out_specs=(
            pl.BlockSpec((2, RB, out_ch // 2), lambda i: (0, i, 0)),
            pl.BlockSpec((NG, out_ch), lambda i: (0, 0)),
            pl.BlockSpec((NG, 1), lambda i: (0, 0)),
        ),
    )(agg2, xm2, xin2, batch_p, rel, root, *extra)


def _out_body(x2_ref, batch_ref, inv8_ref, field_ref, ow_ref,
              ob_ref, out_ref):
    x = jnp.concatenate([x2_ref[0], x2_ref[1]], axis=-1)
    g = _onehot(batch_ref[...], x.shape[0])
    xn = x * jnp.dot(g, inv8_ref[...], precision=lax.Precision.HIGHEST, preferred_element_type=jnp.float32)
    xcat = jnp.concatenate([_silu(xn), field_ref[...]], axis=-1)
    out_ref[...] = jnp.sum(xcat * ow_ref[...], axis=1, keepdims=True) + ob_ref[0, 0]


def _k_out(x2, batch_p, inv8, field_p, out_w, out_b):
    return pl.pallas_call(
        _out_body,
        grid=(GRID,),
        out_shape=jax.ShapeDtypeStruct((NP, 1), jnp.float32),
        in_specs=[
            pl.BlockSpec((2, RB, x2.shape[2]), lambda i: (0, i, 0)),
            pl.BlockSpec((RB, 1), lambda i: (i, 0)),
            pl.BlockSpec(inv8.shape, lambda i: (0, 0)),
            pl.BlockSpec((RB, 1), lambda i: (i, 0)),
            pl.BlockSpec(out_w.shape, lambda i: (0, 0)),
            pl.BlockSpec(memory_space=pltpu.SMEM),
        ],
        out_specs=pl.BlockSpec((RB, 1), lambda i: (i, 0)),
    )(x2, batch_p, inv8, field_p, out_w, out_b.reshape(1, 1))


def _conv_agg(y2, src, dst, ew):
    ch = y2.shape[2] * 2
    y = jnp.concatenate([y2[0], y2[1]], axis=-1)
    msg = y[src] * ew[:, None]
    agg = jax.ops.segment_sum(msg, dst, num_segments=NP)
    c2 = ch // 2
    return jnp.stack([agg[:, :c2], agg[:, c2:]])


def kernel(x_in, edge_index, edge_weight, batch, t, params):
    n = x_in.shape[0]
    pad_n = NP - n
    xin_p = jnp.pad(x_in, ((0, pad_n), (0, 0)))
    batch_p = jnp.pad(batch.astype(jnp.int32), (0, pad_n),
                      constant_values=NG)[:, None]
    field_p = xin_p[:, 1:2]
    src = edge_index[0].astype(jnp.int32)
    dst = edge_index[1].astype(jnp.int32)
    ew = edge_weight

    layer_ps = [params["enc"][0], params["enc"][1], params["latent"],
                params["dec"][0], params["dec"][1]]
    styles = _k_pre(t[:, None], params["time_w1"], params["time_w2"],
                    [p["t_w"] for p in layer_ps],
                    [p["t_b"].reshape(1, -1) for p in layer_ps])

    x2, sums, cnt = _k_in(xin_p, batch_p, params["in_w"])

    def _inv8(s, c):
        return lax.rsqrt(s / jnp.maximum(c, 1.0) + EPS)

    residuals = []
    for li, p in enumerate(layer_ps):
        res2 = None
        if li == 2:
            res2 = residuals[1]
        elif li == 3:
            res2 = residuals[0]
        y2 = _k_a(x2, batch_p, _inv8(sums, cnt), styles[li], field_p, p)
        agg0 = _conv_agg(y2, src, dst, ew)
        xm2 = _k_b(agg0, y2, p["c0_rel"], p["c0_root"])
        agg1 = _conv_agg(xm2, src, dst, ew)
        x2, sums, cnt = _k_c(agg1, xm2, x2, batch_p, p, res2=res2)
        if li < 2:
            residuals.append(x2)

    out_p = _k_out(x2, batch_p, _inv8(sums, cnt), field_p, params["out_w"],
                   params["out_b"])
    return out_p[:n]


# speedup vs baseline: 1.7515x; 1.7515x over previous
import functools
import math

import jax
import jax.numpy as jnp
from jax import lax
from jax.experimental import pallas as pl
from jax.experimental.pallas import tpu as pltpu
from jax.experimental.pallas import tpu_sc as plsc

N_NODES = 50000
NP = 50176
RB = 512
GRID = NP // RB
NG = 8
TDIM = 128
EPS = 1e-6


def _silu(x):
    return x * (1.0 / (1.0 + jnp.exp(-x)))


def _mm_t(x, w, hi=False):
    return lax.dot_general(x, w, (((1,), (1,)), ((), ())),
                           precision=lax.Precision.HIGHEST if hi else None,
                           preferred_element_type=jnp.float32)


def _onehot(b, rows):
    return (b == lax.broadcasted_iota(jnp.int32, (rows, NG), 1)).astype(jnp.float32)


def _pre_body(t_ref, w1_ref, w2_ref, *rest):
    n_layers = (len(rest) - 1) // 3
    tws = rest[:n_layers]
    tbs = rest[n_layers:2 * n_layers]
    cnt_in = rest[2 * n_layers]
    outs = rest[2 * n_layers + 1:]
    half = TDIM // 2
    k = lax.broadcasted_iota(jnp.int32, (half, 1), 0).astype(jnp.float32)
    emb = jnp.exp(k * (-math.log(10000.0) / (half - 1)))
    e = _mm_t(t_ref[...], emb, hi=True)
    e2 = jnp.concatenate([jnp.sin(e), jnp.cos(e)], axis=-1)
    tv = _mm_t(_silu(_mm_t(e2, w1_ref[...])), w2_ref[...])
    st = _silu(tv)
    for i in range(n_layers):
        outs[i][...] = _mm_t(st, tws[i][...]) + tbs[i][...]
    del cnt_in


def _k_pre(t2, w1, w2, tws, tbs):
    nl = len(tws)
    spec_full = lambda a: pl.BlockSpec(a.shape, lambda: tuple(0 for _ in a.shape))
    in_arrays = [t2, w1, w2] + list(tws) + list(tbs) + [jnp.zeros((1, 1), jnp.float32)]
    out_shapes = tuple(jax.ShapeDtypeStruct((NG, tw.shape[0]), jnp.float32) for tw in tws)
    return pl.pallas_call(
        _pre_body,
        out_shape=out_shapes,
        in_specs=[spec_full(a) for a in in_arrays],
        out_specs=tuple(pl.BlockSpec(s.shape, lambda: (0, 0)) for s in out_shapes),
    )(*in_arrays)


def _in_body(xin_ref, batch_ref, inw_ref, y2_ref, sums_ref, cnt_ref):
    x = _mm_t(xin_ref[...], inw_ref[...], hi=True)
    c2 = x.shape[1] // 2
    y2_ref[0] = x[:, :c2]
    y2_ref[1] = x[:, c2:]
    g = _onehot(batch_ref[...], x.shape[0])

    @pl.when(pl.program_id(0) == 0)
    def _():
        sums_ref[...] = jnp.zeros_like(sums_ref)
        cnt_ref[...] = jnp.zeros_like(cnt_ref)

    sums_ref[...] += lax.dot_general(g, x * x, (((0,), (0,)), ((), ())),
                                     precision=lax.Precision.HIGHEST,
                                     preferred_element_type=jnp.float32)
    cnt_ref[...] += jnp.sum(g, axis=0)[:, None]


def _k_in(xin_p, batch_p, in_w):
    ch = in_w.shape[0]
    return pl.pallas_call(
        _in_body,
        grid=(GRID,),
        out_shape=(
            jax.ShapeDtypeStruct((2, NP, ch // 2), jnp.float32),
            jax.ShapeDtypeStruct((NG, ch), jnp.float32),
            jax.ShapeDtypeStruct((NG, 1), jnp.float32),
        ),
        in_specs=[
            pl.BlockSpec((RB, 2), lambda i: (i, 0)),
            pl.BlockSpec((RB, 1), lambda i: (i, 0)),
            pl.BlockSpec(in_w.shape, lambda i: (0, 0)),
        ],
        out_specs=(
            pl.BlockSpec((2, RB, ch // 2), lambda i: (0, i, 0)),
            pl.BlockSpec((NG, ch), lambda i: (0, 0)),
            pl.BlockSpec((NG, 1), lambda i: (0, 0)),
        ),
    )(xin_p, batch_p, in_w)


def _a_body(has_mlp, x2_ref, batch_ref, inv8_ref, style_ref, field_ref,
            fwt_ref, *mlp_refs):
    x = jnp.concatenate([x2_ref[0], x2_ref[1]], axis=-1)
    ch = x.shape[1]
    g = _onehot(batch_ref[...], x.shape[0])
    xn = x * jnp.dot(g, inv8_ref[...], precision=lax.Precision.HIGHEST, preferred_element_type=jnp.float32)
    cond = jnp.dot(g, style_ref[...], precision=lax.Precision.HIGHEST, preferred_element_type=jnp.float32) \
        + _mm_t(field_ref[...], fwt_ref[...], hi=True)
    xf = _silu(xn * (1.0 + cond[:, :ch]) + cond[:, ch:])
    if has_mlp:
        m0w_ref, m0b_ref, m1w_ref, m1b_ref, y2_ref = mlp_refs
        h = _mm_t(_silu(_mm_t(xf, m0w_ref[...]) + m0b_ref[...]), m1w_ref[...]) \
            + m1b_ref[...]
        xf = xf + h
    else:
        (y2_ref,) = mlp_refs
    c2 = ch // 2
    y2_ref[0] = xf[:, :c2]
    y2_ref[1] = xf[:, c2:]


def _k_a(x2, batch_p, inv8, style, field_p, p):
    ch = x2.shape[2] * 2
    has_mlp = "m0_w" in p
    fwt = p["f_w"]
    extra = []
    extra_specs = []
    if has_mlp:
        extra = [p["m0_w"], p["m0_b"].reshape(1, -1), p["m1_w"], p["m1_b"].reshape(1, -1)]
        extra_specs = [pl.BlockSpec(a.shape, lambda i: (0, 0)) for a in extra]
    return pl.pallas_call(
        functools.partial(_a_body, has_mlp),
        grid=(GRID,),
        out_shape=jax.ShapeDtypeStruct((2, NP, ch // 2), jnp.float32),
        in_specs=[
            pl.BlockSpec((2, RB, ch // 2), lambda i: (0, i, 0)),
            pl.BlockSpec((RB, 1), lambda i: (i, 0)),
            pl.BlockSpec(inv8.shape, lambda i: (0, 0)),
            pl.BlockSpec(style.shape, lambda i: (0, 0)),
            pl.BlockSpec((RB, 1), lambda i: (i, 0)),
            pl.BlockSpec(fwt.shape, lambda i: (0, 0)),
        ] + extra_specs,
        out_specs=pl.BlockSpec((2, RB, ch // 2), lambda i: (0, i, 0)),
    )(x2, batch_p, inv8, style, field_p, fwt, *extra)


def _b_body(agg_ref, y_ref, rel_ref, root_ref, out_ref):
    agg = jnp.concatenate([agg_ref[0], agg_ref[1]], axis=-1)
    y = jnp.concatenate([y_ref[0], y_ref[1]], axis=-1)
    xm = _silu(_mm_t(agg, rel_ref[...]) + _mm_t(y, root_ref[...]))
    c2 = xm.shape[1] // 2
    out_ref[0] = xm[:, :c2]
    out_ref[1] = xm[:, c2:]


def _k_b(agg2, y2, rel, root):
    ch = y2.shape[2] * 2
    return pl.pallas_call(
        _b_body,
        grid=(GRID,),
        out_shape=jax.ShapeDtypeStruct((2, NP, ch // 2), jnp.float32),
        in_specs=[
            pl.BlockSpec((2, RB, ch // 2), lambda i: (0, i, 0)),
            pl.BlockSpec((2, RB, ch // 2), lambda i: (0, i, 0)),
            pl.BlockSpec(rel.shape, lambda i: (0, 0)),
            pl.BlockSpec(root.shape, lambda i: (0, 0)),
        ],
        out_specs=pl.BlockSpec((2, RB, ch // 2), lambda i: (0, i, 0)),
    )(agg2, y2, rel, root)


def _c_body(has_sc, has_res, agg_ref, xm_ref, xin_ref, batch_ref, rel_ref,
            root_ref, *rest):
    idx = 0
    sc_ref = None
    res_ref = None
    if has_sc:
        sc_ref = rest[idx]; idx += 1
    if has_res:
        res_ref = rest[idx]; idx += 1
    out2_ref, sums_ref, cnt_ref = rest[idx:]

    agg = jnp.concatenate([agg_ref[0], agg_ref[1]], axis=-1)
    xm = jnp.concatenate([xm_ref[0], xm_ref[1]], axis=-1)
    xin = jnp.concatenate([xin_ref[0], xin_ref[1]], axis=-1)
    out = _mm_t(agg, rel_ref[...]) + _mm_t(xm, root_ref[...])
    if has_sc:
        out = out + _mm_t(xin, sc_ref[...])
    else:
        out = out + xin
    if has_res:
        out = out + jnp.concatenate([res_ref[0], res_ref[1]], axis=-1)

    c2 = out.shape[1] // 2
    out2_ref[0] = out[:, :c2]
    out2_ref[1] = out[:, c2:]

    g = _onehot(batch_ref[...], out.shape[0])

    @pl.when(pl.program_id(0) == 0)
    def _():
        sums_ref[...] = jnp.zeros_like(sums_ref)
        cnt_ref[...] = jnp.zeros_like(cnt_ref)

    sums_ref[...] += lax.dot_general(g, out * out, (((0,), (0,)), ((), ())),
                                     precision=lax.Precision.HIGHEST,
                                     preferred_element_type=jnp.float32)
    cnt_ref[...] += jnp.sum(g, axis=0)[:, None]


def _k_c(agg2, xm2, xin2, batch_p, p, res2=None):
    rel, root = p["c1_rel"], p["c1_root"]
    out_ch = rel.shape[0]
    in_c2 = xm2.shape[2]
    has_sc = "sc_w" in p
    extra = []
    extra_specs = []
    if has_sc:
        extra.append(p["sc_w"])
        extra_specs.append(pl.BlockSpec(p["sc_w"].shape, lambda i: (0, 0)))
    if res2 is not None:
        extra.append(res2)
        extra_specs.append(pl.BlockSpec((2, RB, res2.shape[2]), lambda i: (0, i, 0)))
    return pl.pallas_call(
        functools.partial(_c_body, has_sc, res2 is not None),
        grid=(GRID,),
        out_shape=(
            jax.ShapeDtypeStruct((2, NP, out_ch // 2), jnp.float32),
            jax.ShapeDtypeStruct((NG, out_ch), jnp.float32),
            jax.ShapeDtypeStruct((NG, 1), jnp.float32),
        ),
        in_specs=[
            pl.BlockSpec((2, RB, in_c2), lambda i: (0, i, 0)),
            pl.BlockSpec((2, RB, in_c2), lambda i: (0, i, 0)),
            pl.BlockSpec((2, RB, xin2.shape[2]), lambda i: (0, i, 0)),
            pl.BlockSpec((RB, 1), lambda i: (i, 0)),
            pl.BlockSpec(rel.shape, lambda i: (0, 0)),
            pl.BlockSpec(root.shape, lambda i: (0, 0)),
        ] + extra_specs,
        out_specs=(
            pl.BlockSpec((2, RB, out_ch // 2), lambda i: (0, i, 0)),
            pl.BlockSpec((NG, out_ch), lambda i: (0, 0)),
            pl.BlockSpec((NG, 1), lambda i: (0, 0)),
        ),
    )(agg2, xm2, xin2, batch_p, rel, root, *extra)


def _out_body(x2_ref, batch_ref, inv8_ref, field_ref, ow_ref,
              ob_ref, out_ref):
    x = jnp.concatenate([x2_ref[0], x2_ref[1]], axis=-1)
    g = _onehot(batch_ref[...], x.shape[0])
    xn = x * jnp.dot(g, inv8_ref[...], precision=lax.Precision.HIGHEST, preferred_element_type=jnp.float32)
    xcat = jnp.concatenate([_silu(xn), field_ref[...]], axis=-1)
    out_ref[...] = _mm_t(xcat, ow_ref[...])[:, :1] + ob_ref[0, 0]


def _k_out(x2, batch_p, inv8, field_p, out_w, out_b):
    ow_pad = jnp.concatenate([out_w, jnp.zeros((7, out_w.shape[1]), jnp.float32)], 0)
    return pl.pallas_call(
        _out_body,
        grid=(GRID,),
        out_shape=jax.ShapeDtypeStruct((NP, 1), jnp.float32),
        in_specs=[
            pl.BlockSpec((2, RB, x2.shape[2]), lambda i: (0, i, 0)),
            pl.BlockSpec((RB, 1), lambda i: (i, 0)),
            pl.BlockSpec(inv8.shape, lambda i: (0, 0)),
            pl.BlockSpec((RB, 1), lambda i: (i, 0)),
            pl.BlockSpec((8, out_w.shape[1]), lambda i: (0, 0)),
            pl.BlockSpec(memory_space=pltpu.SMEM),
        ],
        out_specs=pl.BlockSpec((RB, 1), lambda i: (i, 0)),
    )(x2, batch_p, inv8, field_p, ow_pad, out_b.reshape(1, 1))


EP = 800768
EPT = EP // 16
CK = 64
NCH = EPT // CK
TROWS = NP // 16


def _sc_conv_body(c2, xf, srcp, dstp, ewp, zrows, out, acc, src_v, sidx_v,
                  dst_v, ew_v, msg, gsem, ssem):
    c = lax.axis_index("c")
    s = lax.axis_index("s")
    pltpu.sync_copy(zrows, acc.at[pl.ds(s * TROWS, TROWS)])
    plsc.subcore_barrier()

    def chunk(j, carry):
        eb = s * EPT + j * CK
        pltpu.sync_copy(srcp.at[pl.ds(eb, CK)], src_v)
        pltpu.sync_copy(dstp.at[pl.ds(eb, CK)], dst_v)
        pltpu.sync_copy(ewp.at[pl.ds(eb, CK)], ew_v)
        for q in range(CK // 16):
            sidx_v[pl.ds(q * 16, 16)] = src_v[pl.ds(q * 16, 16)] + c * NP
        pltpu.async_copy(xf.at[sidx_v], msg, gsem).wait()
        for eg in range(CK // 16):
            wv = ew_v[pl.ds(eg * 16, 16)]
            for el in range(16):
                e = eg * 16 + el
                w = wv[el]
                for q in range(c2 // 16):
                    msg[e, pl.ds(q * 16, 16)] = msg[e, pl.ds(q * 16, 16)] * w
        pltpu.sync_copy(msg, acc.at[dst_v], add=True)
        return carry

    lax.fori_loop(0, NCH, chunk, 0)
    plsc.subcore_barrier()
    pltpu.sync_copy(acc.at[pl.ds(s * TROWS, TROWS)],
                    out.at[pl.ds(c * NP + s * TROWS, TROWS)])


def _sc_conv(y2, srcp, dstp, ewp):
    c2 = y2.shape[2]
    xf = y2.reshape(2 * NP, c2)
    zrows = jnp.zeros((TROWS, c2), jnp.float32)
    fn = pl.kernel(
        functools.partial(_sc_conv_body, c2),
        out_type=jax.ShapeDtypeStruct((2 * NP, c2), jnp.float32),
        mesh=plsc.VectorSubcoreMesh(core_axis_name="c", subcore_axis_name="s"),
        compiler_params=pltpu.CompilerParams(use_tc_tiling_on_sc=False),
        scratch_types=[
            pltpu.VMEM_SHARED((NP, c2), jnp.float32),
            pltpu.VMEM((CK,), jnp.int32),
            pltpu.VMEM((CK,), jnp.int32),
            pltpu.VMEM((CK,), jnp.int32),
            pltpu.VMEM((CK,), jnp.float32),
            pltpu.VMEM((CK, c2), jnp.float32),
            pltpu.SemaphoreType.DMA,
            pltpu.SemaphoreType.DMA,
        ],
    )
    agg = fn(xf, srcp, dstp, ewp, zrows)
    return agg.reshape(2, NP, c2)


def kernel(x_in, edge_index, edge_weight, batch, t, params):
    n = x_in.shape[0]
    pad_n = NP - n
    xin_p = jnp.pad(x_in, ((0, pad_n), (0, 0)))
    batch_p = jnp.pad(batch.astype(jnp.int32), (0, pad_n),
                      constant_values=NG)[:, None]
    field_p = xin_p[:, 1:2]
    src = edge_index[0].astype(jnp.int32)
    dst = edge_index[1].astype(jnp.int32)
    ew = edge_weight
    perm = jnp.argsort(dst, stable=True)
    src = src[perm]
    dst = dst[perm]
    ew = ew[perm]
    pad_e = EP - src.shape[0]
    srcp = jnp.pad(src, (0, pad_e))
    dstp = jnp.pad(dst, (0, pad_e))
    ewp = jnp.pad(ew, (0, pad_e))

    layer_ps = [params["enc"][0], params["enc"][1], params["latent"],
                params["dec"][0], params["dec"][1]]
    styles = _k_pre(t[:, None], params["time_w1"], params["time_w2"],
                    [p["t_w"] for p in layer_ps],
                    [p["t_b"].reshape(1, -1) for p in layer_ps])

    x2, sums, cnt = _k_in(xin_p, batch_p, params["in_w"])

    def _inv8(s, c):
        return lax.rsqrt(s / jnp.maximum(c, 1.0) + EPS)

    residuals = []
    for li, p in enumerate(layer_ps):
        res2 = None
        if li == 2:
            res2 = residuals[1]
        elif li == 3:
            res2 = residuals[0]
        y2 = _k_a(x2, batch_p, _inv8(sums, cnt), styles[li], field_p, p)
        agg0 = _sc_conv(y2, srcp, dstp, ewp)
        xm2 = _k_b(agg0, y2, p["c0_rel"], p["c0_root"])
        agg1 = _sc_conv(xm2, srcp, dstp, ewp)
        x2, sums, cnt = _k_c(agg1, xm2, x2, batch_p, p, res2=res2)
        if li < 2:
            residuals.append(x2)

    out_p = _k_out(x2, batch_p, _inv8(sums, cnt), field_p, params["out_w"],
                   params["out_b"])
    return out_p[:n]
